# Initial kernel scaffold; baseline (speedup 1.0000x reference)
import jax
import jax.numpy as jnp
from jax.experimental import pallas as pl


def kernel(origins, directions, coarse_density):
    raise NotImplementedError("write your pallas kernel here")


# speedup vs baseline: 126.9313x; 126.9313x over previous
import jax
import jax.numpy as jnp
from jax.experimental import pallas as pl
from jax.experimental.pallas import tpu as pltpu

N_COARSE = 64
N_FINE = 128
N_ALL = N_COARSE + N_FINE
N_PAD = 256
MIN_DEPTH = 2.0
MAX_DEPTH = 6.0
BS = 32
BIG = 3.0e38


def _scan_last(x, op, identity):
    n = x.shape[-1]
    s = 1
    while s < n:
        pad = jnp.full(x.shape[:-1] + (s,), identity, x.dtype)
        x = op(x, jnp.concatenate([pad, x[..., :-s]], axis=-1))
        s *= 2
    return x


def _body(dens_ref, z_ref, u_ref, zea_ref, zeb_ref, o_ref, d_ref,
          zs_ref, pts_ref):
    dens = dens_ref[...]
    z = z_ref[...]
    u = u_ref[...]

    delta = jnp.concatenate(
        [z[:, 1:] - z[:, :-1], jnp.full((BS, 1), 1e10, jnp.float32)], axis=-1)
    alpha = 1.0 - jnp.exp(-dens * delta)
    am = 1.0 - alpha + 1e-10
    cp = _scan_last(am, jnp.multiply, jnp.float32(1.0))
    T = jnp.concatenate(
        [jnp.ones((BS, 1), jnp.float32), cp[:, :-1]], axis=-1)
    w = alpha * T + 1e-05
    pdf = w / jnp.sum(w, axis=-1, keepdims=True)
    csum = _scan_last(pdf, jnp.add, jnp.float32(0.0))
    zea = zea_ref[...]
    zeb = zeb_ref[...]

    cdf0 = jnp.zeros((BS, N_FINE), jnp.float32)
    e0 = jnp.broadcast_to(zea[:, :1], (BS, N_FINE))
    cdf1 = jnp.broadcast_to(csum[:, N_COARSE - 1:], (BS, N_FINE))
    e1 = jnp.broadcast_to(zeb[:, N_COARSE - 1:], (BS, N_FINE))

    for j in range(N_COARSE):
        cs_j = csum[:, j:j + 1]
        ze_j = zeb[:, j:j + 1]
        le = u >= cs_j
        cdf0 = jnp.where(le, cs_j, cdf0)
        e0 = jnp.where(le, ze_j, e0)
        j2 = N_COARSE - 1 - j
        cs_j2 = csum[:, j2:j2 + 1]
        ze_j2 = zeb[:, j2:j2 + 1]
        gt = cs_j2 > u
        cdf1 = jnp.where(gt, cs_j2, cdf1)
        e1 = jnp.where(gt, ze_j2, e1)

    denom = cdf1 - cdf0
    denom = jnp.where(denom < 1e-05, jnp.float32(1.0), denom)
    t = (u - cdf0) / denom
    fine = e0 + t * (e1 - e0)

    s = jnp.concatenate(
        [z, jnp.full((BS, N_COARSE), BIG, jnp.float32), fine], axis=-1)
    lane = jax.lax.broadcasted_iota(jnp.int32, (BS, N_PAD), 1)
    step_sz = N_PAD // 2
    while step_sz >= 1:
        up = jnp.concatenate([s[:, step_sz:], s[:, :step_sz]], axis=-1)
        dn = jnp.concatenate([s[:, -step_sz:], s[:, :-step_sz]], axis=-1)
        is_lo = (lane & step_sz) == 0
        s = jnp.where(is_lo, jnp.minimum(s, up), jnp.maximum(s, dn))
        step_sz //= 2
    zs = s[:, :N_ALL]
    zs_ref[...] = zs

    o = o_ref[...]
    d = d_ref[...]
    pts_ref[...] = o[:, :, None] + d[:, :, None] * zs[:, None, :]


def kernel(origins, directions, coarse_density):
    B = origins.shape[0]
    with jax.ensure_compile_time_eval():
        rkey = jax.random.key(12345)
        rk1, rk2 = jax.random.split(rkey)
        zlin = jnp.linspace(MIN_DEPTH, MAX_DEPTH, N_COARSE, dtype=jnp.float32)
        z_vals = jnp.broadcast_to(zlin, (B, N_COARSE))
        mids = 0.5 * (z_vals[..., 1:] + z_vals[..., :-1])
        upper = jnp.concatenate([mids, z_vals[..., -1:]], axis=-1)
        lower = jnp.concatenate([z_vals[..., :1], mids], axis=-1)
        t_rand = jax.random.uniform(rk1, z_vals.shape, dtype=jnp.float32)
        z_vals = lower + (upper - lower) * t_rand
        u = jax.random.uniform(rk2, (B, N_FINE), dtype=jnp.float32)
        u_desc = -jnp.sort(-u, axis=-1)
        ze_mid = 0.5 * (z_vals[..., 1:] + z_vals[..., :-1])
        z_edges = jnp.concatenate(
            [
                z_vals[..., :1] - 0.5 * (z_vals[..., 1:2] - z_vals[..., :1]),
                ze_mid,
                z_vals[..., -1:] + 0.5 * (z_vals[..., -1:] - z_vals[..., -2:-1]),
            ],
            axis=-1,
        )
        zea = z_edges[:, :N_COARSE]
        zeb = z_edges[:, 1:]

    dens = coarse_density[..., 0]

    grid = (B // BS,)
    zs, pts = pl.pallas_call(
        _body,
        grid=grid,
        in_specs=[
            pl.BlockSpec((BS, N_COARSE), lambda b: (b, 0)),
            pl.BlockSpec((BS, N_COARSE), lambda b: (b, 0)),
            pl.BlockSpec((BS, N_FINE), lambda b: (b, 0)),
            pl.BlockSpec((BS, N_COARSE), lambda b: (b, 0)),
            pl.BlockSpec((BS, N_COARSE), lambda b: (b, 0)),
            pl.BlockSpec((BS, 3), lambda b: (b, 0)),
            pl.BlockSpec((BS, 3), lambda b: (b, 0)),
        ],
        out_specs=[
            pl.BlockSpec((BS, N_ALL), lambda b: (b, 0)),
            pl.BlockSpec((BS, 3, N_ALL), lambda b: (b, 0, 0)),
        ],
        out_shape=[
            jax.ShapeDtypeStruct((B, N_ALL), jnp.float32),
            jax.ShapeDtypeStruct((B, 3, N_ALL), jnp.float32),
        ],
    )(dens, z_vals, u_desc, zea, zeb, origins, directions)

    return (pts.transpose(0, 2, 1), zs[..., None])


# speedup vs baseline: 193.8842x; 1.5275x over previous
import jax
import jax.numpy as jnp
from jax.experimental import pallas as pl
from jax.experimental.pallas import tpu as pltpu

N_COARSE = 64
N_FINE = 128
N_ALL = N_COARSE + N_FINE
N_PAD = 256
MIN_DEPTH = 2.0
MAX_DEPTH = 6.0
BS = 64
BIG = 3.0e38


def _scan_last(x, op, identity):
    n = x.shape[-1]
    s = 1
    while s < n:
        pad = jnp.full(x.shape[:-1] + (s,), identity, x.dtype)
        x = op(x, jnp.concatenate([pad, x[..., :-s]], axis=-1))
        s *= 2
    return x


def _body(dens_ref, z_ref, u_ref, zea_ref, zeb_ref, o_ref, d_ref,
          zs_ref, pts_ref):
    dens = dens_ref[...]
    z = z_ref[...]
    u = u_ref[...]

    delta = jnp.concatenate(
        [z[:, 1:] - z[:, :-1], jnp.full((BS, 1), 1e10, jnp.float32)], axis=-1)
    alpha = 1.0 - jnp.exp(-dens * delta)
    am = 1.0 - alpha + 1e-10
    cp = _scan_last(am, jnp.multiply, jnp.float32(1.0))
    T = jnp.concatenate(
        [jnp.ones((BS, 1), jnp.float32), cp[:, :-1]], axis=-1)
    w = alpha * T + 1e-05
    pdf = w / jnp.sum(w, axis=-1, keepdims=True)
    csum = _scan_last(pdf, jnp.add, jnp.float32(0.0))
    zea = zea_ref[...]
    zeb = zeb_ref[...]

    cdf0 = jnp.zeros((BS, N_FINE), jnp.float32)
    e0 = jnp.broadcast_to(zea[:, :1], (BS, N_FINE))
    cdf1 = jnp.broadcast_to(csum[:, N_COARSE - 1:], (BS, N_FINE))
    e1 = jnp.broadcast_to(zeb[:, N_COARSE - 1:], (BS, N_FINE))

    for j in range(N_COARSE):
        cs_j = csum[:, j:j + 1]
        ze_j = zeb[:, j:j + 1]
        le = u >= cs_j
        cdf0 = jnp.where(le, cs_j, cdf0)
        e0 = jnp.where(le, ze_j, e0)
        j2 = N_COARSE - 1 - j
        cs_j2 = csum[:, j2:j2 + 1]
        ze_j2 = zeb[:, j2:j2 + 1]
        gt = cs_j2 > u
        cdf1 = jnp.where(gt, cs_j2, cdf1)
        e1 = jnp.where(gt, ze_j2, e1)

    denom = cdf1 - cdf0
    denom = jnp.where(denom < 1e-05, jnp.float32(1.0), denom)
    t = (u - cdf0) / denom
    fine = e0 + t * (e1 - e0)

    s = jnp.concatenate(
        [z, jnp.full((BS, N_COARSE), BIG, jnp.float32), fine], axis=-1)
    lane = jax.lax.broadcasted_iota(jnp.int32, (BS, N_PAD), 1)
    step_sz = N_PAD // 2
    while step_sz >= 1:
        up = jnp.concatenate([s[:, step_sz:], s[:, :step_sz]], axis=-1)
        dn = jnp.concatenate([s[:, -step_sz:], s[:, :-step_sz]], axis=-1)
        is_lo = (lane & step_sz) == 0
        s = jnp.where(is_lo, jnp.minimum(s, up), jnp.maximum(s, dn))
        step_sz //= 2
    zs = s[:, :N_ALL]
    zs_ref[...] = zs

    o = o_ref[...]
    d = d_ref[...]
    pts_ref[...] = o[:, :, None] + d[:, :, None] * zs[:, None, :]


def kernel(origins, directions, coarse_density):
    B = origins.shape[0]
    with jax.ensure_compile_time_eval():
        rkey = jax.random.key(12345)
        rk1, rk2 = jax.random.split(rkey)
        zlin = jnp.linspace(MIN_DEPTH, MAX_DEPTH, N_COARSE, dtype=jnp.float32)
        z_vals = jnp.broadcast_to(zlin, (B, N_COARSE))
        mids = 0.5 * (z_vals[..., 1:] + z_vals[..., :-1])
        upper = jnp.concatenate([mids, z_vals[..., -1:]], axis=-1)
        lower = jnp.concatenate([z_vals[..., :1], mids], axis=-1)
        t_rand = jax.random.uniform(rk1, z_vals.shape, dtype=jnp.float32)
        z_vals = lower + (upper - lower) * t_rand
        u = jax.random.uniform(rk2, (B, N_FINE), dtype=jnp.float32)
        u_desc = -jnp.sort(-u, axis=-1)
        ze_mid = 0.5 * (z_vals[..., 1:] + z_vals[..., :-1])
        z_edges = jnp.concatenate(
            [
                z_vals[..., :1] - 0.5 * (z_vals[..., 1:2] - z_vals[..., :1]),
                ze_mid,
                z_vals[..., -1:] + 0.5 * (z_vals[..., -1:] - z_vals[..., -2:-1]),
            ],
            axis=-1,
        )
        zea = z_edges[:, :N_COARSE]
        zeb = z_edges[:, 1:]

    dens = coarse_density[..., 0]

    grid = (B // BS,)
    zs, pts = pl.pallas_call(
        _body,
        grid=grid,
        in_specs=[
            pl.BlockSpec((BS, N_COARSE), lambda b: (b, 0)),
            pl.BlockSpec((BS, N_COARSE), lambda b: (b, 0)),
            pl.BlockSpec((BS, N_FINE), lambda b: (b, 0)),
            pl.BlockSpec((BS, N_COARSE), lambda b: (b, 0)),
            pl.BlockSpec((BS, N_COARSE), lambda b: (b, 0)),
            pl.BlockSpec((BS, 3), lambda b: (b, 0)),
            pl.BlockSpec((BS, 3), lambda b: (b, 0)),
        ],
        out_specs=[
            pl.BlockSpec((BS, N_ALL), lambda b: (b, 0)),
            pl.BlockSpec((BS, 3, N_ALL), lambda b: (b, 0, 0)),
        ],
        out_shape=[
            jax.ShapeDtypeStruct((B, N_ALL), jnp.float32),
            jax.ShapeDtypeStruct((B, 3, N_ALL), jnp.float32),
        ],
    )(dens, z_vals, u_desc, zea, zeb, origins, directions)

    return (pts.transpose(0, 2, 1), zs[..., None])


# speedup vs baseline: 228.5150x; 1.1786x over previous
import jax
import jax.numpy as jnp
from jax.experimental import pallas as pl
from jax.experimental.pallas import tpu as pltpu

N_COARSE = 64
N_FINE = 128
N_ALL = N_COARSE + N_FINE
N_PAD = 256
MIN_DEPTH = 2.0
MAX_DEPTH = 6.0
BS = 128
BIG = 3.0e38


def _scan_last(x, op, identity):
    n = x.shape[-1]
    s = 1
    while s < n:
        pad = jnp.full(x.shape[:-1] + (s,), identity, x.dtype)
        x = op(x, jnp.concatenate([pad, x[..., :-s]], axis=-1))
        s *= 2
    return x


def _body(dens_ref, z_ref, u_ref, zea_ref, zeb_ref, o_ref, d_ref,
          zs_ref, pts_ref):
    dens = dens_ref[...]
    z = z_ref[...]
    u = u_ref[...]

    delta = jnp.concatenate(
        [z[:, 1:] - z[:, :-1], jnp.full((BS, 1), 1e10, jnp.float32)], axis=-1)
    alpha = 1.0 - jnp.exp(-dens * delta)
    am = 1.0 - alpha + 1e-10
    cp = _scan_last(am, jnp.multiply, jnp.float32(1.0))
    T = jnp.concatenate(
        [jnp.ones((BS, 1), jnp.float32), cp[:, :-1]], axis=-1)
    w = alpha * T + 1e-05
    pdf = w / jnp.sum(w, axis=-1, keepdims=True)
    csum = _scan_last(pdf, jnp.add, jnp.float32(0.0))
    zea = zea_ref[...]
    zeb = zeb_ref[...]

    cdf0 = jnp.zeros((BS, N_FINE), jnp.float32)
    e0 = jnp.broadcast_to(zea[:, :1], (BS, N_FINE))
    cdf1 = jnp.broadcast_to(csum[:, N_COARSE - 1:], (BS, N_FINE))
    e1 = jnp.broadcast_to(zeb[:, N_COARSE - 1:], (BS, N_FINE))

    for j in range(N_COARSE):
        cs_j = csum[:, j:j + 1]
        ze_j = zeb[:, j:j + 1]
        le = u >= cs_j
        cdf0 = jnp.where(le, cs_j, cdf0)
        e0 = jnp.where(le, ze_j, e0)
        j2 = N_COARSE - 1 - j
        cs_j2 = csum[:, j2:j2 + 1]
        ze_j2 = zeb[:, j2:j2 + 1]
        gt = cs_j2 > u
        cdf1 = jnp.where(gt, cs_j2, cdf1)
        e1 = jnp.where(gt, ze_j2, e1)

    denom = cdf1 - cdf0
    denom = jnp.where(denom < 1e-05, jnp.float32(1.0), denom)
    t = (u - cdf0) / denom
    fine = e0 + t * (e1 - e0)

    s = jnp.concatenate(
        [z, jnp.full((BS, N_COARSE), BIG, jnp.float32), fine], axis=-1)
    lane = jax.lax.broadcasted_iota(jnp.int32, (BS, N_PAD), 1)
    step_sz = N_PAD // 2
    while step_sz >= 1:
        up = jnp.concatenate([s[:, step_sz:], s[:, :step_sz]], axis=-1)
        dn = jnp.concatenate([s[:, -step_sz:], s[:, :-step_sz]], axis=-1)
        is_lo = (lane & step_sz) == 0
        s = jnp.where(is_lo, jnp.minimum(s, up), jnp.maximum(s, dn))
        step_sz //= 2
    zs = s[:, :N_ALL]
    zs_ref[...] = zs

    o = o_ref[...]
    d = d_ref[...]
    pts_ref[...] = o[:, :, None] + d[:, :, None] * zs[:, None, :]


def kernel(origins, directions, coarse_density):
    B = origins.shape[0]
    with jax.ensure_compile_time_eval():
        rkey = jax.random.key(12345)
        rk1, rk2 = jax.random.split(rkey)
        zlin = jnp.linspace(MIN_DEPTH, MAX_DEPTH, N_COARSE, dtype=jnp.float32)
        z_vals = jnp.broadcast_to(zlin, (B, N_COARSE))
        mids = 0.5 * (z_vals[..., 1:] + z_vals[..., :-1])
        upper = jnp.concatenate([mids, z_vals[..., -1:]], axis=-1)
        lower = jnp.concatenate([z_vals[..., :1], mids], axis=-1)
        t_rand = jax.random.uniform(rk1, z_vals.shape, dtype=jnp.float32)
        z_vals = lower + (upper - lower) * t_rand
        u = jax.random.uniform(rk2, (B, N_FINE), dtype=jnp.float32)
        u_desc = -jnp.sort(-u, axis=-1)
        ze_mid = 0.5 * (z_vals[..., 1:] + z_vals[..., :-1])
        z_edges = jnp.concatenate(
            [
                z_vals[..., :1] - 0.5 * (z_vals[..., 1:2] - z_vals[..., :1]),
                ze_mid,
                z_vals[..., -1:] + 0.5 * (z_vals[..., -1:] - z_vals[..., -2:-1]),
            ],
            axis=-1,
        )
        zea = z_edges[:, :N_COARSE]
        zeb = z_edges[:, 1:]

    dens = coarse_density[..., 0]

    grid = (B // BS,)
    zs, pts = pl.pallas_call(
        _body,
        grid=grid,
        in_specs=[
            pl.BlockSpec((BS, N_COARSE), lambda b: (b, 0)),
            pl.BlockSpec((BS, N_COARSE), lambda b: (b, 0)),
            pl.BlockSpec((BS, N_FINE), lambda b: (b, 0)),
            pl.BlockSpec((BS, N_COARSE), lambda b: (b, 0)),
            pl.BlockSpec((BS, N_COARSE), lambda b: (b, 0)),
            pl.BlockSpec((BS, 3), lambda b: (b, 0)),
            pl.BlockSpec((BS, 3), lambda b: (b, 0)),
        ],
        out_specs=[
            pl.BlockSpec((BS, N_ALL), lambda b: (b, 0)),
            pl.BlockSpec((BS, 3, N_ALL), lambda b: (b, 0, 0)),
        ],
        out_shape=[
            jax.ShapeDtypeStruct((B, N_ALL), jnp.float32),
            jax.ShapeDtypeStruct((B, 3, N_ALL), jnp.float32),
        ],
    )(dens, z_vals, u_desc, zea, zeb, origins, directions)

    return (pts.transpose(0, 2, 1), zs[..., None])


# speedup vs baseline: 243.8635x; 1.0672x over previous
import jax
import jax.numpy as jnp
from jax.experimental import pallas as pl
from jax.experimental.pallas import tpu as pltpu

N_COARSE = 64
N_FINE = 128
N_ALL = N_COARSE + N_FINE
N_PAD = 256
MIN_DEPTH = 2.0
MAX_DEPTH = 6.0
BS = 256
BIG = 3.0e38


def _scan_last(x, op, identity):
    n = x.shape[-1]
    s = 1
    while s < n:
        pad = jnp.full(x.shape[:-1] + (s,), identity, x.dtype)
        x = op(x, jnp.concatenate([pad, x[..., :-s]], axis=-1))
        s *= 2
    return x


def _body(dens_ref, z_ref, u_ref, zea_ref, zeb_ref, o_ref, d_ref,
          zs_ref, pts_ref):
    dens = dens_ref[...]
    z = z_ref[...]
    u = u_ref[...]

    delta = jnp.concatenate(
        [z[:, 1:] - z[:, :-1], jnp.full((BS, 1), 1e10, jnp.float32)], axis=-1)
    alpha = 1.0 - jnp.exp(-dens * delta)
    am = 1.0 - alpha + 1e-10
    cp = _scan_last(am, jnp.multiply, jnp.float32(1.0))
    T = jnp.concatenate(
        [jnp.ones((BS, 1), jnp.float32), cp[:, :-1]], axis=-1)
    w = alpha * T + 1e-05
    pdf = w / jnp.sum(w, axis=-1, keepdims=True)
    csum = _scan_last(pdf, jnp.add, jnp.float32(0.0))
    zea = zea_ref[...]
    zeb = zeb_ref[...]

    cdf0 = jnp.zeros((BS, N_FINE), jnp.float32)
    e0 = jnp.broadcast_to(zea[:, :1], (BS, N_FINE))
    cdf1 = jnp.broadcast_to(csum[:, N_COARSE - 1:], (BS, N_FINE))
    e1 = jnp.broadcast_to(zeb[:, N_COARSE - 1:], (BS, N_FINE))

    for j in range(N_COARSE):
        cs_j = csum[:, j:j + 1]
        ze_j = zeb[:, j:j + 1]
        le = u >= cs_j
        cdf0 = jnp.where(le, cs_j, cdf0)
        e0 = jnp.where(le, ze_j, e0)
        j2 = N_COARSE - 1 - j
        cs_j2 = csum[:, j2:j2 + 1]
        ze_j2 = zeb[:, j2:j2 + 1]
        gt = cs_j2 > u
        cdf1 = jnp.where(gt, cs_j2, cdf1)
        e1 = jnp.where(gt, ze_j2, e1)

    denom = cdf1 - cdf0
    denom = jnp.where(denom < 1e-05, jnp.float32(1.0), denom)
    t = (u - cdf0) / denom
    fine = e0 + t * (e1 - e0)

    s = jnp.concatenate(
        [z, jnp.full((BS, N_COARSE), BIG, jnp.float32), fine], axis=-1)
    lane = jax.lax.broadcasted_iota(jnp.int32, (BS, N_PAD), 1)
    step_sz = N_PAD // 2
    while step_sz >= 1:
        up = jnp.concatenate([s[:, step_sz:], s[:, :step_sz]], axis=-1)
        dn = jnp.concatenate([s[:, -step_sz:], s[:, :-step_sz]], axis=-1)
        is_lo = (lane & step_sz) == 0
        s = jnp.where(is_lo, jnp.minimum(s, up), jnp.maximum(s, dn))
        step_sz //= 2
    zs = s[:, :N_ALL]
    zs_ref[...] = zs

    o = o_ref[...]
    d = d_ref[...]
    pts_ref[...] = o[:, :, None] + d[:, :, None] * zs[:, None, :]


def kernel(origins, directions, coarse_density):
    B = origins.shape[0]
    with jax.ensure_compile_time_eval():
        rkey = jax.random.key(12345)
        rk1, rk2 = jax.random.split(rkey)
        zlin = jnp.linspace(MIN_DEPTH, MAX_DEPTH, N_COARSE, dtype=jnp.float32)
        z_vals = jnp.broadcast_to(zlin, (B, N_COARSE))
        mids = 0.5 * (z_vals[..., 1:] + z_vals[..., :-1])
        upper = jnp.concatenate([mids, z_vals[..., -1:]], axis=-1)
        lower = jnp.concatenate([z_vals[..., :1], mids], axis=-1)
        t_rand = jax.random.uniform(rk1, z_vals.shape, dtype=jnp.float32)
        z_vals = lower + (upper - lower) * t_rand
        u = jax.random.uniform(rk2, (B, N_FINE), dtype=jnp.float32)
        u_desc = -jnp.sort(-u, axis=-1)
        ze_mid = 0.5 * (z_vals[..., 1:] + z_vals[..., :-1])
        z_edges = jnp.concatenate(
            [
                z_vals[..., :1] - 0.5 * (z_vals[..., 1:2] - z_vals[..., :1]),
                ze_mid,
                z_vals[..., -1:] + 0.5 * (z_vals[..., -1:] - z_vals[..., -2:-1]),
            ],
            axis=-1,
        )
        zea = z_edges[:, :N_COARSE]
        zeb = z_edges[:, 1:]

    dens = coarse_density[..., 0]

    grid = (B // BS,)
    zs, pts = pl.pallas_call(
        _body,
        grid=grid,
        in_specs=[
            pl.BlockSpec((BS, N_COARSE), lambda b: (b, 0)),
            pl.BlockSpec((BS, N_COARSE), lambda b: (b, 0)),
            pl.BlockSpec((BS, N_FINE), lambda b: (b, 0)),
            pl.BlockSpec((BS, N_COARSE), lambda b: (b, 0)),
            pl.BlockSpec((BS, N_COARSE), lambda b: (b, 0)),
            pl.BlockSpec((BS, 3), lambda b: (b, 0)),
            pl.BlockSpec((BS, 3), lambda b: (b, 0)),
        ],
        out_specs=[
            pl.BlockSpec((BS, N_ALL), lambda b: (b, 0)),
            pl.BlockSpec((BS, 3, N_ALL), lambda b: (b, 0, 0)),
        ],
        out_shape=[
            jax.ShapeDtypeStruct((B, N_ALL), jnp.float32),
            jax.ShapeDtypeStruct((B, 3, N_ALL), jnp.float32),
        ],
    )(dens, z_vals, u_desc, zea, zeb, origins, directions)

    return (pts.transpose(0, 2, 1), zs[..., None])


# speedup vs baseline: 248.8300x; 1.0204x over previous
import jax
import jax.numpy as jnp
from jax.experimental import pallas as pl
from jax.experimental.pallas import tpu as pltpu

N_COARSE = 64
N_FINE = 128
N_ALL = N_COARSE + N_FINE
N_PAD = 256
MIN_DEPTH = 2.0
MAX_DEPTH = 6.0
BS = 512
BIG = 3.0e38


def _scan_last(x, op, identity):
    n = x.shape[-1]
    s = 1
    while s < n:
        pad = jnp.full(x.shape[:-1] + (s,), identity, x.dtype)
        x = op(x, jnp.concatenate([pad, x[..., :-s]], axis=-1))
        s *= 2
    return x


def _body(dens_ref, z_ref, u_ref, zea_ref, zeb_ref, o_ref, d_ref,
          zs_ref, pts_ref):
    dens = dens_ref[...]
    z = z_ref[...]
    u = u_ref[...]

    delta = jnp.concatenate(
        [z[:, 1:] - z[:, :-1], jnp.full((BS, 1), 1e10, jnp.float32)], axis=-1)
    alpha = 1.0 - jnp.exp(-dens * delta)
    am = 1.0 - alpha + 1e-10
    cp = _scan_last(am, jnp.multiply, jnp.float32(1.0))
    T = jnp.concatenate(
        [jnp.ones((BS, 1), jnp.float32), cp[:, :-1]], axis=-1)
    w = alpha * T + 1e-05
    pdf = w / jnp.sum(w, axis=-1, keepdims=True)
    csum = _scan_last(pdf, jnp.add, jnp.float32(0.0))
    zea = zea_ref[...]
    zeb = zeb_ref[...]

    cdf0 = jnp.zeros((BS, N_FINE), jnp.float32)
    e0 = jnp.broadcast_to(zea[:, :1], (BS, N_FINE))
    cdf1 = jnp.broadcast_to(csum[:, N_COARSE - 1:], (BS, N_FINE))
    e1 = jnp.broadcast_to(zeb[:, N_COARSE - 1:], (BS, N_FINE))

    for j in range(N_COARSE):
        cs_j = csum[:, j:j + 1]
        ze_j = zeb[:, j:j + 1]
        le = u >= cs_j
        cdf0 = jnp.where(le, cs_j, cdf0)
        e0 = jnp.where(le, ze_j, e0)
        j2 = N_COARSE - 1 - j
        cs_j2 = csum[:, j2:j2 + 1]
        ze_j2 = zeb[:, j2:j2 + 1]
        gt = cs_j2 > u
        cdf1 = jnp.where(gt, cs_j2, cdf1)
        e1 = jnp.where(gt, ze_j2, e1)

    denom = cdf1 - cdf0
    denom = jnp.where(denom < 1e-05, jnp.float32(1.0), denom)
    t = (u - cdf0) / denom
    fine = e0 + t * (e1 - e0)

    s = jnp.concatenate(
        [z, jnp.full((BS, N_COARSE), BIG, jnp.float32), fine], axis=-1)
    lane = jax.lax.broadcasted_iota(jnp.int32, (BS, N_PAD), 1)
    step_sz = N_PAD // 2
    while step_sz >= 1:
        up = jnp.concatenate([s[:, step_sz:], s[:, :step_sz]], axis=-1)
        dn = jnp.concatenate([s[:, -step_sz:], s[:, :-step_sz]], axis=-1)
        is_lo = (lane & step_sz) == 0
        s = jnp.where(is_lo, jnp.minimum(s, up), jnp.maximum(s, dn))
        step_sz //= 2
    zs = s[:, :N_ALL]
    zs_ref[...] = zs

    o = o_ref[...]
    d = d_ref[...]
    pts_ref[...] = o[:, :, None] + d[:, :, None] * zs[:, None, :]


def kernel(origins, directions, coarse_density):
    B = origins.shape[0]
    with jax.ensure_compile_time_eval():
        rkey = jax.random.key(12345)
        rk1, rk2 = jax.random.split(rkey)
        zlin = jnp.linspace(MIN_DEPTH, MAX_DEPTH, N_COARSE, dtype=jnp.float32)
        z_vals = jnp.broadcast_to(zlin, (B, N_COARSE))
        mids = 0.5 * (z_vals[..., 1:] + z_vals[..., :-1])
        upper = jnp.concatenate([mids, z_vals[..., -1:]], axis=-1)
        lower = jnp.concatenate([z_vals[..., :1], mids], axis=-1)
        t_rand = jax.random.uniform(rk1, z_vals.shape, dtype=jnp.float32)
        z_vals = lower + (upper - lower) * t_rand
        u = jax.random.uniform(rk2, (B, N_FINE), dtype=jnp.float32)
        u_desc = -jnp.sort(-u, axis=-1)
        ze_mid = 0.5 * (z_vals[..., 1:] + z_vals[..., :-1])
        z_edges = jnp.concatenate(
            [
                z_vals[..., :1] - 0.5 * (z_vals[..., 1:2] - z_vals[..., :1]),
                ze_mid,
                z_vals[..., -1:] + 0.5 * (z_vals[..., -1:] - z_vals[..., -2:-1]),
            ],
            axis=-1,
        )
        zea = z_edges[:, :N_COARSE]
        zeb = z_edges[:, 1:]

    dens = coarse_density[..., 0]

    grid = (B // BS,)
    zs, pts = pl.pallas_call(
        _body,
        grid=grid,
        in_specs=[
            pl.BlockSpec((BS, N_COARSE), lambda b: (b, 0)),
            pl.BlockSpec((BS, N_COARSE), lambda b: (b, 0)),
            pl.BlockSpec((BS, N_FINE), lambda b: (b, 0)),
            pl.BlockSpec((BS, N_COARSE), lambda b: (b, 0)),
            pl.BlockSpec((BS, N_COARSE), lambda b: (b, 0)),
            pl.BlockSpec((BS, 3), lambda b: (b, 0)),
            pl.BlockSpec((BS, 3), lambda b: (b, 0)),
        ],
        out_specs=[
            pl.BlockSpec((BS, N_ALL), lambda b: (b, 0)),
            pl.BlockSpec((BS, 3, N_ALL), lambda b: (b, 0, 0)),
        ],
        out_shape=[
            jax.ShapeDtypeStruct((B, N_ALL), jnp.float32),
            jax.ShapeDtypeStruct((B, 3, N_ALL), jnp.float32),
        ],
    )(dens, z_vals, u_desc, zea, zeb, origins, directions)

    return (pts.transpose(0, 2, 1), zs[..., None])
